# baseline (speedup 1.0000x reference)
import functools

import jax
import jax.numpy as jnp
from jax import lax
from jax.experimental import pallas as pl
from jax.experimental.pallas import tpu as pltpu
from jax.experimental.pallas import tpu_sc as plsc

_PADDING_IDX = 0
_SMOOTHING = 0.1
_CONFIDENCE = 1.0 - _SMOOTHING


def _sc_gather_target_rows(x_rows, target_flat, vocab):
    n = target_flat.shape[0]
    info = plsc.get_sparse_core_info()
    nc, ns, lanes = info.num_cores, info.num_subcores, info.num_lanes
    nw = nc * ns
    n_per_w = n // nw
    mesh = plsc.VectorSubcoreMesh(core_axis_name="c", subcore_axis_name="s")

    @functools.partial(
        pl.kernel,
        mesh=mesh,
        out_type=jax.ShapeDtypeStruct((n, 128), jnp.float32),
        scratch_types=[
            pltpu.VMEM((n_per_w,), jnp.int32),
            pltpu.VMEM((n_per_w,), jnp.int32),
            pltpu.VMEM((n_per_w, 128), jnp.float32),
            pltpu.SemaphoreType.DMA,
        ],
    )
    def gather_kernel(x_hbm, tgt_hbm, out_hbm, t_v, r_v, rows_v, sem):
        wid = lax.axis_index("s") * nc + lax.axis_index("c")
        base = wid * n_per_w
        pltpu.sync_copy(tgt_hbm.at[pl.ds(base, n_per_w)], t_v)
        for j in range(n_per_w // lanes):
            t = t_v[pl.ds(j * lanes, lanes)]
            tok = lax.iota(jnp.int32, lanes) + (base + j * lanes)
            r_v[pl.ds(j * lanes, lanes)] = tok * (vocab // 128) + lax.shift_right_logical(t, 7)
        pltpu.async_copy(x_hbm.at[r_v], rows_v, sem).wait()
        pltpu.sync_copy(rows_v, out_hbm.at[pl.ds(base, n_per_w)])

    return gather_kernel(x_rows, target_flat)


def _tc_loss(x2, tgt3, rows3, vocab):
    rows = x2.shape[0]
    block_rows = 256
    grid = rows // block_rows
    e_k = _SMOOTHING / (vocab - 2)

    def body(x_ref, t_ref, r_ref, out_ref):
        i = pl.program_id(0)
        rs = jnp.sum(x_ref[...], axis=1)
        t = t_ref[0, 0, :]
        gr = r_ref[0, :, :]
        c = lax.bitwise_and(t, 127)
        lane = lax.broadcasted_iota(jnp.int32, (block_rows, 128), 1)
        g = jnp.sum(jnp.where(lane == c[:, None], gr, 0.0), axis=1)
        per = jnp.where(t != _PADDING_IDX, e_k * rs + (_CONFIDENCE - e_k) * g, 0.0)

        @pl.when(i == 0)
        def _():
            out_ref[...] = jnp.zeros_like(out_ref)

        out_ref[...] += jnp.reshape(-jnp.sum(per), (1, 1))

    out = pl.pallas_call(
        body,
        grid=(grid,),
        in_specs=[
            pl.BlockSpec((block_rows, vocab), lambda i: (i, 0)),
            pl.BlockSpec((1, 1, block_rows), lambda i: (i, 0, 0)),
            pl.BlockSpec((1, block_rows, 128), lambda i: (i, 0, 0)),
        ],
        out_specs=pl.BlockSpec((1, 1), lambda i: (0, 0)),
        out_shape=jax.ShapeDtypeStruct((1, 1), jnp.float32),
    )(x2, tgt3, rows3)
    return out[0, 0]


def kernel(x, target):
    b, l, v = x.shape
    r = b * l
    block_rows = 256
    x2 = x.reshape(r, v)
    tflat = target.reshape(r)
    grows = _sc_gather_target_rows(x.reshape(r * v // 128, 128), tflat, v)
    return _tc_loss(
        x2,
        tflat.reshape(r // block_rows, 1, block_rows),
        grows.reshape(r // block_rows, block_rows, 128),
        v,
    )


# speedup vs baseline: 3.0584x; 3.0584x over previous
import functools

import jax
import jax.numpy as jnp
from jax import lax
from jax.experimental import pallas as pl
from jax.experimental.pallas import tpu as pltpu
from jax.experimental.pallas import tpu_sc as plsc

_PADDING_IDX = 0
_SMOOTHING = 0.1
_CONFIDENCE = 1.0 - _SMOOTHING


def _sc_gather_target_rows(x2, target_flat):
    n = target_flat.shape[0]
    info = plsc.get_sparse_core_info()
    nc, ns = info.num_cores, info.num_subcores
    nw = nc * ns
    n_per_w = n // nw
    mesh = plsc.VectorSubcoreMesh(core_axis_name="c", subcore_axis_name="s")

    chunk = 32

    @functools.partial(
        pl.kernel,
        mesh=mesh,
        out_type=jax.ShapeDtypeStruct((n, 128), jnp.float32),
        scratch_types=[
            pltpu.VMEM((n_per_w,), jnp.int32),
            pltpu.VMEM((2, chunk, 8, 128), jnp.float32),
            pltpu.VMEM((n_per_w, 128), jnp.float32),
            pltpu.SemaphoreType.DMA,
            pltpu.SemaphoreType.DMA,
        ],
    )
    def gather_kernel(x_hbm, tgt_hbm, out_hbm, t_v, tiles_v, rows_v, sem0, sem1):
        wid = lax.axis_index("s") * nc + lax.axis_index("c")
        base = wid * n_per_w
        pltpu.sync_copy(tgt_hbm.at[pl.ds(base, n_per_w)], t_v)
        sems = (sem0, sem1)

        def fire(k):
            copies = []
            for jo in range(chunk // 16):
                t_vec = t_v[pl.ds(k * chunk + jo * 16, 16)]
                cbs = lax.shift_left(lax.shift_right_logical(t_vec, 7), 7)
                for ji in range(16):
                    j = k * chunk + jo * 16 + ji
                    copies.append(
                        pltpu.async_copy(
                            x_hbm.at[
                                pl.ds(base + (j & ~7), 8),
                                pl.ds(pl.multiple_of(cbs[ji], 128), 128),
                            ],
                            tiles_v.at[k & 1, j - k * chunk],
                            sems[k & 1],
                        )
                    )
            return copies

        pending = fire(0)
        for k in range(n_per_w // chunk):
            nxt = fire(k + 1) if (k + 1) < n_per_w // chunk else []
            for c in pending:
                c.wait()
            pending = nxt
            for jj in range(chunk):
                j = k * chunk + jj
                for c8 in range(8):
                    rows_v[j, pl.ds(c8 * 16, 16)] = tiles_v[k & 1, jj, j & 7, pl.ds(c8 * 16, 16)]
        pltpu.sync_copy(rows_v, out_hbm.at[pl.ds(base, n_per_w)])

    return gather_kernel(x2, target_flat)


def _tc_loss(x2, tgt3, rows3, vocab):
    rows = x2.shape[0]
    block_rows = 256
    grid = rows // block_rows
    e_k = _SMOOTHING / (vocab - 2)

    def body(x_ref, t_ref, r_ref, out_ref):
        i = pl.program_id(0)
        rs = jnp.sum(x_ref[...], axis=1)
        t = t_ref[0, 0, :]
        gr = r_ref[0, :, :]
        c = lax.bitwise_and(t, 127)
        lane = lax.broadcasted_iota(jnp.int32, (block_rows, 128), 1)
        g = jnp.sum(jnp.where(lane == c[:, None], gr, 0.0), axis=1)
        per = jnp.where(t != _PADDING_IDX, e_k * rs + (_CONFIDENCE - e_k) * g, 0.0)

        @pl.when(i == 0)
        def _():
            out_ref[...] = jnp.zeros_like(out_ref)

        out_ref[...] += jnp.reshape(-jnp.sum(per), (1, 1))

    out = pl.pallas_call(
        body,
        grid=(grid,),
        in_specs=[
            pl.BlockSpec((block_rows, vocab), lambda i: (i, 0)),
            pl.BlockSpec((1, 1, block_rows), lambda i: (i, 0, 0)),
            pl.BlockSpec((1, block_rows, 128), lambda i: (i, 0, 0)),
        ],
        out_specs=pl.BlockSpec((1, 1), lambda i: (0, 0)),
        out_shape=jax.ShapeDtypeStruct((1, 1), jnp.float32),
    )(x2, tgt3, rows3)
    return out[0, 0]


def kernel(x, target):
    b, l, v = x.shape
    r = b * l
    block_rows = 256
    x2 = x.reshape(r, v)
    tflat = target.reshape(r)
    grows = _sc_gather_target_rows(x2, tflat)
    return _tc_loss(
        x2,
        tflat.reshape(r // block_rows, 1, block_rows),
        grows.reshape(r // block_rows, block_rows, 128),
        v,
    )
